# Initial kernel scaffold; baseline (speedup 1.0000x reference)
import jax
import jax.numpy as jnp
from jax.experimental import pallas as pl


def kernel(features, g, task, W_pred, b_pred):
    raise NotImplementedError("write your pallas kernel here")


# speedup vs baseline: 8.8920x; 8.8920x over previous
import functools

import jax
import jax.numpy as jnp
from jax import lax
from jax.experimental import pallas as pl
from jax.experimental.pallas import tpu as pltpu
from jax.experimental.pallas import tpu_sc as plsc

NC = 2
NS = 16
NW = NC * NS
CH = 128


def _matmul_body(f_ref, w_ref, o_ref, *, deg_col):
    q = jnp.dot(f_ref[...], w_ref[...], preferred_element_type=jnp.float32)
    col = lax.broadcasted_iota(jnp.int32, q.shape, 1)
    o_ref[...] = q + (col == deg_col).astype(jnp.float32)


def _combine_body(p_ref, q_ref, b_ref, o_ref, *, n_cls):
    pp = p_ref[0] + p_ref[1]
    deg = jnp.maximum(pp[:, n_cls:n_cls + 1], 1.0)
    o_ref[...] = pp[:, :n_cls] / deg + q_ref[:, :n_cls] + b_ref[...]


def _make_sc_segsum(rpad, cpad, ept, nchunk):
    rpt = rpad // NS
    mesh = plsc.VectorSubcoreMesh(core_axis_name="c", subcore_axis_name="s")

    @functools.partial(
        pl.kernel,
        mesh=mesh,
        compiler_params=pltpu.CompilerParams(use_tc_tiling_on_sc=False),
        out_type=jax.ShapeDtypeStruct((NC * rpad, cpad), jnp.float32),
        scratch_types=[
            pltpu.VMEM((ept,), jnp.int32),
            pltpu.VMEM((CH,), jnp.int32),
            pltpu.VMEM((CH,), jnp.int32),
            pltpu.VMEM((CH, cpad), jnp.float32),
            pltpu.VMEM((CH, cpad), jnp.float32),
            pltpu.VMEM_SHARED((rpad, cpad), jnp.float32),
            pltpu.SemaphoreType.DMA,
            pltpu.SemaphoreType.DMA,
            pltpu.SemaphoreType.DMA,
            pltpu.SemaphoreType.DMA,
        ],
    )
    def segsum(q_hbm, src_hbm, dst_hbm, zeros_hbm, out_hbm,
               src_v, dst0, dst1, rows0, rows1, acc,
               dsem0, dsem1, gsem0, gsem1):
        cid = lax.axis_index("c")
        sid = lax.axis_index("s")
        wid = cid * NS + sid
        base = wid * ept
        dst_r = (dst0, dst1)
        rows_r = (rows0, rows1)
        dsem = (dsem0, dsem1)
        gsem = (gsem0, gsem1)

        pltpu.sync_copy(zeros_hbm.at[pl.ds(sid * rpt, rpt)],
                        acc.at[pl.ds(sid * rpt, rpt)])
        pltpu.sync_copy(src_hbm.at[pl.ds(base, ept)], src_v)

        def prime(i, s):
            pltpu.async_copy(dst_hbm.at[pl.ds(base + i * CH, CH)],
                             dst_r[s], dsem[s])
            pltpu.async_copy(q_hbm.at[src_v.at[pl.ds(i * CH, CH)]],
                             rows_r[s], gsem[s])

        prime(0, 0)
        prime(1, 1)
        plsc.subcore_barrier()

        def body(j, carry):
            for s in range(2):
                i = 2 * j + s
                pltpu.make_async_copy(
                    dst_hbm.at[pl.ds(base + i * CH, CH)], dst_r[s],
                    dsem[s]).wait()
                pltpu.make_async_copy(
                    q_hbm.at[src_v.at[pl.ds(i * CH, CH)]], rows_r[s],
                    gsem[s]).wait()
                pltpu.sync_copy(rows_r[s], acc.at[dst_r[s]], add=True)

                @pl.when(i + 2 < nchunk)
                def _():
                    prime(i + 2, s)
            return carry

        lax.fori_loop(0, nchunk // 2, body, 0)

        plsc.subcore_barrier()
        pltpu.sync_copy(acc.at[pl.ds(sid * rpt, rpt)],
                        out_hbm.at[pl.ds(cid * rpad + sid * rpt, rpt)])

    return segsum


def kernel(features, g, task, W_pred, b_pred):
    n, d = features.shape
    n_cls = W_pred.shape[1]
    e = g.shape[1]
    del task

    cpad = 48
    rpad = ((n + NS * 8 - 1) // (NS * 8)) * (NS * 8)
    if rpad == n:
        rpad += NS * 8
    nchunk = -(-e // (NW * CH))
    nchunk += nchunk % 2
    ept = nchunk * CH
    e_pad = NW * ept
    bm = rpad // 8
    bmc = n // 10

    f_pad = jnp.pad(features, ((0, rpad - n), (0, 0)))
    w_pad = jnp.pad(W_pred, ((0, 0), (0, cpad - n_cls)))
    pad_e = e_pad - e
    src_p = jnp.concatenate([g[0], jnp.zeros((pad_e,), jnp.int32)])
    dst_p = jnp.concatenate(
        [g[1], n + (jnp.arange(pad_e, dtype=jnp.int32) % (rpad - n))])
    zeros = jnp.zeros((rpad, cpad), jnp.float32)

    q = pl.pallas_call(
        functools.partial(_matmul_body, deg_col=n_cls),
        grid=(rpad // bm,),
        in_specs=[pl.BlockSpec((bm, d), lambda i: (i, 0)),
                  pl.BlockSpec((d, cpad), lambda i: (0, 0))],
        out_specs=pl.BlockSpec((bm, cpad), lambda i: (i, 0)),
        out_shape=jax.ShapeDtypeStruct((rpad, cpad), jnp.float32),
    )(f_pad, w_pad)

    partials = _make_sc_segsum(rpad, cpad, ept, nchunk)(q, src_p, dst_p, zeros)
    partials = partials.reshape(NC, rpad, cpad)

    logits = pl.pallas_call(
        functools.partial(_combine_body, n_cls=n_cls),
        grid=(n // bmc,),
        in_specs=[pl.BlockSpec((NC, bmc, cpad), lambda i: (0, i, 0)),
                  pl.BlockSpec((bmc, cpad), lambda i: (i, 0)),
                  pl.BlockSpec((1, n_cls), lambda i: (0, 0))],
        out_specs=pl.BlockSpec((bmc, n_cls), lambda i: (i, 0)),
        out_shape=jax.ShapeDtypeStruct((n, n_cls), jnp.float32),
    )(partials, q, b_pred.reshape(1, n_cls))
    return logits


# speedup vs baseline: 20.7656x; 2.3353x over previous
import functools

import jax
import jax.numpy as jnp
from jax import lax
from jax.experimental import pallas as pl
from jax.experimental.pallas import tpu as pltpu
from jax.experimental.pallas import tpu_sc as plsc

NC = 2
NS = 16
NW = NC * NS
CH = 128
NSLOT = 4


def _matmul_body(f_ref, w_ref, o_ref, *, deg_col):
    q = jnp.dot(f_ref[...], w_ref[...], preferred_element_type=jnp.float32)
    col = lax.broadcasted_iota(jnp.int32, q.shape, 1)
    o_ref[...] = q + (col == deg_col).astype(jnp.float32)


def _combine_body(p0_ref, p1_ref, q_ref, b_ref, o_ref, *, n_cls):
    pp = p0_ref[...] + p1_ref[...]
    deg = jnp.maximum(pp[:, n_cls:n_cls + 1], 1.0)
    o_ref[...] = pp[:, :n_cls] / deg + q_ref[:, :n_cls] + b_ref[...]


def _make_sc_segsum(n, cpad, e):
    total_chunks = e // CH
    stage = -(-total_chunks // NW) * CH
    rpt = n // NS
    mesh = plsc.VectorSubcoreMesh(core_axis_name="c", subcore_axis_name="s")

    @functools.partial(
        pl.kernel,
        mesh=mesh,
        compiler_params=pltpu.CompilerParams(use_tc_tiling_on_sc=False),
        out_type=jax.ShapeDtypeStruct((NC * n, cpad), jnp.float32),
        scratch_types=(
            [pltpu.VMEM((stage,), jnp.int32)]
            + [pltpu.VMEM((CH,), jnp.int32) for _ in range(NSLOT)]
            + [pltpu.VMEM((CH, cpad), jnp.float32) for _ in range(NSLOT)]
            + [pltpu.VMEM_SHARED((n, cpad), jnp.float32)]
            + [pltpu.SemaphoreType.DMA for _ in range(3 * NSLOT)]
        ),
    )
    def segsum(q_hbm, g_hbm, zeros_hbm, out_hbm, src_v,
               d0, d1, d2, d3, r0, r1, r2, r3, acc, *sems):
        cid = lax.axis_index("c")
        sid = lax.axis_index("s")
        wid = cid * NS + sid
        dst_r = (d0, d1, d2, d3)
        rows_r = (r0, r1, r2, r3)
        dsem = sems[0:NSLOT]
        gsem = sems[NSLOT:2 * NSLOT]
        ssem = sems[2 * NSLOT:3 * NSLOT]

        c0 = wid * total_chunks // NW
        nk = (wid + 1) * total_chunks // NW - c0
        base = c0 * CH

        pltpu.sync_copy(zeros_hbm.at[pl.ds(sid * rpt, rpt)],
                        acc.at[pl.ds(sid * rpt, rpt)])
        pltpu.sync_copy(g_hbm.at[0, pl.ds(base, stage)], src_v)

        def issue_front(k, s):
            pltpu.async_copy(g_hbm.at[1, pl.ds(base + k * CH, CH)],
                             dst_r[s], dsem[s])
            pltpu.async_copy(q_hbm.at[src_v.at[pl.ds(k * CH, CH)]],
                             rows_r[s], gsem[s])

        def wait_front(k, s):
            pltpu.make_async_copy(g_hbm.at[1, pl.ds(base + k * CH, CH)],
                                  dst_r[s], dsem[s]).wait()
            pltpu.make_async_copy(q_hbm.at[src_v.at[pl.ds(k * CH, CH)]],
                                  rows_r[s], gsem[s]).wait()

        def issue_scatter(s):
            pltpu.async_copy(rows_r[s], acc.at[dst_r[s]], ssem[s], add=True)

        def wait_scatter(s):
            pltpu.make_async_copy(rows_r[s], acc.at[dst_r[s]],
                                  ssem[s]).wait()

        for s in range(NSLOT):
            @pl.when(s < nk)
            def _(s=s):
                issue_front(s, s)

        plsc.subcore_barrier()

        def body(j, carry):
            for s in range(NSLOT):
                i = NSLOT * j + s
                wait_front(i, s)
                issue_scatter(s)
            for s in range(NSLOT):
                nxt = NSLOT * j + NSLOT + s

                @pl.when(nxt < nk)
                def _(s=s, nxt=nxt):
                    wait_scatter(s)
                    issue_front(nxt, s)
            return carry

        lax.fori_loop(0, nk // NSLOT, body, 0)

        tail_base = (nk // NSLOT) * NSLOT
        for s in range(NSLOT):
            @pl.when(tail_base + s < nk)
            def _(s=s, i=tail_base + s):
                wait_front(i, s)
                issue_scatter(s)
        for s in range(NSLOT):
            @pl.when(s < nk)
            def _(s=s):
                wait_scatter(s)

        plsc.subcore_barrier()
        pltpu.sync_copy(acc.at[pl.ds(sid * rpt, rpt)],
                        out_hbm.at[pl.ds(cid * n + sid * rpt, rpt)])

    return segsum


def kernel(features, g, task, W_pred, b_pred):
    n, d = features.shape
    n_cls = W_pred.shape[1]
    e = g.shape[1]
    del task

    cpad = 48
    bm = 1000

    w_pad = jnp.pad(W_pred, ((0, 0), (0, cpad - n_cls)))
    zeros = jnp.zeros((n, cpad), jnp.float32)

    q = pl.pallas_call(
        functools.partial(_matmul_body, deg_col=n_cls),
        grid=(n // bm,),
        in_specs=[pl.BlockSpec((bm, d), lambda i: (i, 0)),
                  pl.BlockSpec((d, cpad), lambda i: (0, 0))],
        out_specs=pl.BlockSpec((bm, cpad), lambda i: (i, 0)),
        out_shape=jax.ShapeDtypeStruct((n, cpad), jnp.float32),
    )(features, w_pad)

    partials = _make_sc_segsum(n, cpad, e)(q, g, zeros)

    nb = n // bm
    logits = pl.pallas_call(
        functools.partial(_combine_body, n_cls=n_cls),
        grid=(nb,),
        in_specs=[pl.BlockSpec((bm, cpad), lambda i: (i, 0)),
                  pl.BlockSpec((bm, cpad), lambda i: (i + nb, 0)),
                  pl.BlockSpec((bm, cpad), lambda i: (i, 0)),
                  pl.BlockSpec((1, n_cls), lambda i: (0, 0))],
        out_specs=pl.BlockSpec((bm, n_cls), lambda i: (i, 0)),
        out_shape=jax.ShapeDtypeStruct((n, n_cls), jnp.float32),
    )(partials, partials, q, b_pred.reshape(1, n_cls))
    return logits


# speedup vs baseline: 22.3638x; 1.0770x over previous
import functools

import jax
import jax.numpy as jnp
from jax import lax
from jax.experimental import pallas as pl
from jax.experimental.pallas import tpu as pltpu
from jax.experimental.pallas import tpu_sc as plsc

NC = 2
NS = 16
NW = NC * NS
CH = 128
NSLOT = 4


def _matmul_body(f_ref, w_ref, o_ref, *, deg_col):
    q = jnp.dot(f_ref[...], w_ref[...], preferred_element_type=jnp.float32)
    col = lax.broadcasted_iota(jnp.int32, q.shape, 1)
    o_ref[...] = q + (col == deg_col).astype(jnp.float32)


def _combine_body(p0_ref, p1_ref, q_ref, b_ref, o_ref, *, n_cls):
    pp = p0_ref[...] + p1_ref[...]
    deg = jnp.maximum(pp[:, n_cls:n_cls + 1], 1.0)
    o_ref[...] = pp[:, :n_cls] / deg + q_ref[:, :n_cls] + b_ref[...]


def _make_sc_segsum(n, cpad, e):
    total_chunks = e // CH
    stage = -(-total_chunks // NW) * CH
    rpt = n // NS
    mesh = plsc.VectorSubcoreMesh(core_axis_name="c", subcore_axis_name="s")

    @functools.partial(
        pl.kernel,
        mesh=mesh,
        compiler_params=pltpu.CompilerParams(use_tc_tiling_on_sc=False),
        out_type=jax.ShapeDtypeStruct((NC * n, 128), jnp.float32),
        scratch_types=(
            [pltpu.VMEM((stage,), jnp.int32)]
            + [pltpu.VMEM((CH,), jnp.int32) for _ in range(NSLOT)]
            + [pltpu.VMEM((CH, cpad), jnp.float32) for _ in range(NSLOT)]
            + [pltpu.VMEM_SHARED((n, cpad), jnp.float32)]
            + [pltpu.SemaphoreType.DMA for _ in range(3 * NSLOT)]
        ),
    )
    def segsum(q_hbm, g_hbm, zeros_hbm, out_hbm, src_v,
               d0, d1, d2, d3, r0, r1, r2, r3, acc, *sems):
        cid = lax.axis_index("c")
        sid = lax.axis_index("s")
        wid = cid * NS + sid
        dst_r = (d0, d1, d2, d3)
        rows_r = (r0, r1, r2, r3)
        dsem = sems[0:NSLOT]
        gsem = sems[NSLOT:2 * NSLOT]
        ssem = sems[2 * NSLOT:3 * NSLOT]

        c0 = wid * total_chunks // NW
        nk = (wid + 1) * total_chunks // NW - c0
        base = c0 * CH

        pltpu.sync_copy(zeros_hbm.at[pl.ds(sid * rpt, rpt)],
                        acc.at[pl.ds(sid * rpt, rpt)])
        pltpu.sync_copy(g_hbm.at[0, pl.ds(base, stage)], src_v)

        def issue_front(k, s):
            pltpu.async_copy(g_hbm.at[1, pl.ds(base + k * CH, CH)],
                             dst_r[s], dsem[s])
            pltpu.async_copy(q_hbm.at[src_v.at[pl.ds(k * CH, CH)]],
                             rows_r[s], gsem[s])

        def wait_front(k, s):
            pltpu.make_async_copy(g_hbm.at[1, pl.ds(base + k * CH, CH)],
                                  dst_r[s], dsem[s]).wait()
            pltpu.make_async_copy(q_hbm.at[src_v.at[pl.ds(k * CH, CH)]],
                                  rows_r[s], gsem[s]).wait()

        def issue_scatter(s):
            pltpu.async_copy(rows_r[s], acc.at[dst_r[s]], ssem[s], add=True)

        def wait_scatter(s):
            pltpu.make_async_copy(rows_r[s], acc.at[dst_r[s]],
                                  ssem[s]).wait()

        for s in range(NSLOT):
            @pl.when(s < nk)
            def _(s=s):
                issue_front(s, s)

        plsc.subcore_barrier()

        def body(j, carry):
            for s in range(NSLOT):
                i = NSLOT * j + s
                wait_front(i, s)
                issue_scatter(s)
            for s in range(NSLOT):
                nxt = NSLOT * j + NSLOT + s

                @pl.when(nxt < nk)
                def _(s=s, nxt=nxt):
                    wait_scatter(s)
                    issue_front(nxt, s)
            return carry

        lax.fori_loop(0, nk // NSLOT, body, 0)

        tail_base = (nk // NSLOT) * NSLOT
        for s in range(NSLOT):
            @pl.when(tail_base + s < nk)
            def _(s=s, i=tail_base + s):
                wait_front(i, s)
                issue_scatter(s)
        for s in range(NSLOT):
            @pl.when(s < nk)
            def _(s=s):
                wait_scatter(s)

        plsc.subcore_barrier()
        pltpu.sync_copy(acc.at[pl.ds(sid * rpt, rpt)],
                        out_hbm.at[pl.ds(cid * n + sid * rpt, rpt),
                                   pl.ds(0, cpad)])

    return segsum


def kernel(features, g, task, W_pred, b_pred):
    n, d = features.shape
    n_cls = W_pred.shape[1]
    e = g.shape[1]
    del task

    cpad = 48
    bm = 1000

    w_pad = jnp.pad(W_pred, ((0, 0), (0, cpad - n_cls)))
    zeros = jnp.zeros((n, cpad), jnp.float32)

    q = pl.pallas_call(
        functools.partial(_matmul_body, deg_col=n_cls),
        grid=(n // bm,),
        in_specs=[pl.BlockSpec((bm, d), lambda i: (i, 0)),
                  pl.BlockSpec((d, cpad), lambda i: (0, 0))],
        out_specs=pl.BlockSpec((bm, cpad), lambda i: (i, 0)),
        out_shape=jax.ShapeDtypeStruct((n, cpad), jnp.float32),
    )(features, w_pad)

    partials = _make_sc_segsum(n, cpad, e)(q, g, zeros)

    nb = n // bm
    logits = pl.pallas_call(
        functools.partial(_combine_body, n_cls=n_cls),
        grid=(nb,),
        in_specs=[pl.BlockSpec((bm, 128), lambda i: (i, 0)),
                  pl.BlockSpec((bm, 128), lambda i: (i + nb, 0)),
                  pl.BlockSpec((bm, cpad), lambda i: (i, 0)),
                  pl.BlockSpec((1, n_cls), lambda i: (0, 0))],
        out_specs=pl.BlockSpec((bm, n_cls), lambda i: (i, 0)),
        out_shape=jax.ShapeDtypeStruct((n, n_cls), jnp.float32),
    )(partials, partials, q, b_pred.reshape(1, n_cls))
    return logits


# speedup vs baseline: 23.4545x; 1.0488x over previous
import functools

import jax
import jax.numpy as jnp
from jax import lax
from jax.experimental import pallas as pl
from jax.experimental.pallas import tpu as pltpu
from jax.experimental.pallas import tpu_sc as plsc

NC = 2
NS = 16
NW = NC * NS
CH = 128
NSLOT = 8


def _matmul_body(f_ref, w_ref, o_ref, *, deg_col):
    q = jnp.dot(f_ref[...], w_ref[...], preferred_element_type=jnp.float32)
    col = lax.broadcasted_iota(jnp.int32, q.shape, 1)
    o_ref[...] = q + (col == deg_col).astype(jnp.float32)


def _combine_body(p0_ref, p1_ref, q_ref, b_ref, o_ref, *, n_cls):
    pp = p0_ref[...] + p1_ref[...]
    deg = jnp.maximum(pp[:, n_cls:n_cls + 1], 1.0)
    o_ref[...] = pp[:, :n_cls] / deg + q_ref[:, :n_cls] + b_ref[...]


def _make_sc_segsum(n, cpad, e):
    total_chunks = e // CH
    stage = -(-total_chunks // NW) * CH
    rpt = n // NS
    mesh = plsc.VectorSubcoreMesh(core_axis_name="c", subcore_axis_name="s")

    @functools.partial(
        pl.kernel,
        mesh=mesh,
        compiler_params=pltpu.CompilerParams(use_tc_tiling_on_sc=False),
        out_type=jax.ShapeDtypeStruct((NC * n, 128), jnp.float32),
        scratch_types=(
            [pltpu.VMEM((stage,), jnp.int32)]
            + [pltpu.VMEM((CH,), jnp.int32) for _ in range(NSLOT)]
            + [pltpu.VMEM((CH, cpad), jnp.float32) for _ in range(NSLOT)]
            + [pltpu.VMEM_SHARED((n, cpad), jnp.float32)]
            + [pltpu.SemaphoreType.DMA for _ in range(3 * NSLOT)]
        ),
    )
    def segsum(q_hbm, g_hbm, out_hbm, src_v, *rest):
        cid = lax.axis_index("c")
        sid = lax.axis_index("s")
        wid = cid * NS + sid
        dst_r = rest[0:NSLOT]
        rows_r = rest[NSLOT:2 * NSLOT]
        acc = rest[2 * NSLOT]
        sems = rest[2 * NSLOT + 1:]
        dsem = sems[0:NSLOT]
        gsem = sems[NSLOT:2 * NSLOT]
        ssem = sems[2 * NSLOT:3 * NSLOT]

        c0 = wid * total_chunks // NW
        nk = (wid + 1) * total_chunks // NW - c0
        base = c0 * CH

        zrow = jnp.zeros((16,), jnp.float32)

        def zloop(i, carry):
            for c in range(cpad // 16):
                rows_r[0][i, pl.ds(16 * c, 16)] = zrow
            return carry

        lax.fori_loop(0, CH, zloop, 0)
        nfull, tailr = rpt // CH, rpt % CH
        for z in range(nfull):
            pltpu.sync_copy(rows_r[0],
                            acc.at[pl.ds(sid * rpt + z * CH, CH)])
        if tailr:
            pltpu.sync_copy(rows_r[0].at[pl.ds(0, tailr)],
                            acc.at[pl.ds(sid * rpt + nfull * CH, tailr)])
        pltpu.sync_copy(g_hbm.at[0, pl.ds(base, stage)], src_v)

        def issue_front(k, s):
            pltpu.async_copy(g_hbm.at[1, pl.ds(base + k * CH, CH)],
                             dst_r[s], dsem[s])
            pltpu.async_copy(q_hbm.at[src_v.at[pl.ds(k * CH, CH)]],
                             rows_r[s], gsem[s])

        def wait_front(k, s):
            pltpu.make_async_copy(g_hbm.at[1, pl.ds(base + k * CH, CH)],
                                  dst_r[s], dsem[s]).wait()
            pltpu.make_async_copy(q_hbm.at[src_v.at[pl.ds(k * CH, CH)]],
                                  rows_r[s], gsem[s]).wait()

        def issue_scatter(s):
            pltpu.async_copy(rows_r[s], acc.at[dst_r[s]], ssem[s], add=True)

        def wait_scatter(s):
            pltpu.make_async_copy(rows_r[s], acc.at[dst_r[s]],
                                  ssem[s]).wait()

        for s in range(NSLOT):
            @pl.when(s < nk)
            def _(s=s):
                issue_front(s, s)

        plsc.subcore_barrier()

        def body(j, carry):
            for s in range(NSLOT):
                i = NSLOT * j + s
                wait_front(i, s)
                issue_scatter(s)
            for s in range(NSLOT):
                nxt = NSLOT * j + NSLOT + s

                @pl.when(nxt < nk)
                def _(s=s, nxt=nxt):
                    wait_scatter(s)
                    issue_front(nxt, s)
            return carry

        lax.fori_loop(0, nk // NSLOT, body, 0)

        tail_base = (nk // NSLOT) * NSLOT
        for s in range(NSLOT):
            @pl.when(tail_base + s < nk)
            def _(s=s, i=tail_base + s):
                wait_front(i, s)
                issue_scatter(s)
        for s in range(NSLOT):
            @pl.when(s < nk)
            def _(s=s):
                wait_scatter(s)

        plsc.subcore_barrier()
        pltpu.sync_copy(acc.at[pl.ds(sid * rpt, rpt)],
                        out_hbm.at[pl.ds(cid * n + sid * rpt, rpt),
                                   pl.ds(0, cpad)])

    return segsum


def kernel(features, g, task, W_pred, b_pred):
    n, d = features.shape
    n_cls = W_pred.shape[1]
    e = g.shape[1]
    del task

    cpad = 48
    bm = 1000

    w_pad = jnp.pad(W_pred, ((0, 0), (0, cpad - n_cls)))

    q = pl.pallas_call(
        functools.partial(_matmul_body, deg_col=n_cls),
        grid=(n // bm,),
        in_specs=[pl.BlockSpec((bm, d), lambda i: (i, 0)),
                  pl.BlockSpec((d, cpad), lambda i: (0, 0))],
        out_specs=pl.BlockSpec((bm, cpad), lambda i: (i, 0)),
        out_shape=jax.ShapeDtypeStruct((n, cpad), jnp.float32),
    )(features, w_pad)

    partials = _make_sc_segsum(n, cpad, e)(q, g)

    nb = n // bm
    logits = pl.pallas_call(
        functools.partial(_combine_body, n_cls=n_cls),
        grid=(nb,),
        in_specs=[pl.BlockSpec((bm, 128), lambda i: (i, 0)),
                  pl.BlockSpec((bm, 128), lambda i: (i + nb, 0)),
                  pl.BlockSpec((bm, cpad), lambda i: (i, 0)),
                  pl.BlockSpec((1, n_cls), lambda i: (0, 0))],
        out_specs=pl.BlockSpec((bm, n_cls), lambda i: (i, 0)),
        out_shape=jax.ShapeDtypeStruct((n, n_cls), jnp.float32),
    )(partials, partials, q, b_pred.reshape(1, n_cls))
    return logits
